# baseline (speedup 1.0000x reference)
import functools

import numpy as np

import jax
import jax.numpy as jnp
from jax import lax
from jax.experimental import pallas as pl
from jax.experimental.pallas import tpu as pltpu
from jax.experimental.pallas import tpu_sc as plsc

N = 10000
E = 160000
D = 16
H = 128
DD = D * D

NC = 2
NS = 16
NW = NC * NS
EPW = E // NW
CH = 125
NCH = EPW // CH
NP = 10240
RPS = NP // NS
NPK = NP // 8
NK = N // 8

EB = 32000
GR = EB // 8
NB = E // EB
EK = E // 8

_MESH = plsc.VectorSubcoreMesh(core_axis_name="c", subcore_axis_name="s")
_SC_PARAMS = pltpu.CompilerParams(use_tc_tiling_on_sc=False)


def _fill_rows(ref, n, value):
    def body(i, carry):
        ref[i, :] = jnp.full((D,), value, jnp.float32)
        return carry
    lax.fori_loop(0, n, body, 0)


def _fire_drain(n, fire):
    def fire_body(j, carry):
        fire(j)
        return carry
    lax.fori_loop(0, n, fire_body, 0)

    def drain_body(j, carry):
        fire(0, wait=True)
        return carry
    lax.fori_loop(0, n, drain_body, 0)


@functools.partial(
    pl.kernel,
    out_type=jax.ShapeDtypeStruct((NC, NP, D), jnp.float32),
    scratch_types=[
        pltpu.VMEM((NCH, CH), jnp.int32),
        pltpu.VMEM((CH, D), jnp.float32),
        pltpu.VMEM((RPS, D), jnp.float32),
        pltpu.SemaphoreType.DMA,
        pltpu.VMEM_SHARED((NP, D), jnp.float32),
    ],
    mesh=_MESH,
    compiler_params=_SC_PARAMS,
)
def _sc_counts(dst_hbm, cnt_out, didx, obuf, zbuf, csem, cnt_sh):
    c = lax.axis_index("c")
    s = lax.axis_index("s")
    wid = s * NC + c
    pltpu.sync_copy(dst_hbm.at[pl.ds(wid * NCH, NCH)], didx)
    _fill_rows(obuf, CH, 1.0)
    _fill_rows(zbuf, RPS, 0.0)
    pltpu.sync_copy(zbuf, cnt_sh.at[pl.ds(s * RPS, RPS)])
    plsc.subcore_barrier()

    def cadd(j, wait=False):
        d = pltpu.make_async_copy(obuf, cnt_sh.at[didx.at[j]], csem)
        d.wait() if wait else d.start(add=True)
    _fire_drain(NCH, cadd)

    plsc.subcore_barrier()
    pltpu.sync_copy(cnt_sh.at[pl.ds(s * RPS, RPS)],
                    cnt_out.at[c].at[pl.ds(s * RPS, RPS)])


def _make_gather(ep):
    epw = ep // NW
    nch = epw // CH

    @functools.partial(
        pl.kernel,
        out_type=jax.ShapeDtypeStruct((ep, D), jnp.float32),
        scratch_types=[
            pltpu.VMEM((nch, CH), jnp.int32),
            pltpu.VMEM((epw, D), jnp.float32),
            pltpu.SemaphoreType.DMA,
        ],
        mesh=_MESH,
        compiler_params=_SC_PARAMS,
    )
    def gather(x_hbm, src_hbm, xs_out, sidx, rows, gsem):
        c = lax.axis_index("c")
        s = lax.axis_index("s")
        wid = s * NC + c
        pltpu.sync_copy(src_hbm.at[pl.ds(wid * nch, nch)], sidx)

        def gath(j, wait=False):
            d = pltpu.make_async_copy(x_hbm.at[sidx.at[j]],
                                      rows.at[pl.ds(j * CH, CH)], gsem)
            d.wait() if wait else d.start()
        _fire_drain(nch, gath)

        pltpu.sync_copy(rows, xs_out.at[pl.ds(wid * epw, epw)])
    return gather


def _make_scatter(ep):
    epw = ep // NW
    nch = epw // CH

    @functools.partial(
        pl.kernel,
        out_type=jax.ShapeDtypeStruct((NC, NP, D), jnp.float32),
        scratch_types=[
            pltpu.VMEM((nch, CH), jnp.int32),
            pltpu.VMEM((epw, D), jnp.float32),
            pltpu.VMEM((RPS, D), jnp.float32),
            pltpu.SemaphoreType.DMA,
            pltpu.VMEM_SHARED((NP, D), jnp.float32),
        ],
        mesh=_MESH,
        compiler_params=_SC_PARAMS,
    )
    def scatter(dst_hbm, msg_hbm, agg_out, didx, buf, zbuf, asem, agg_sh):
        c = lax.axis_index("c")
        s = lax.axis_index("s")
        wid = s * NC + c
        pltpu.sync_copy(dst_hbm.at[pl.ds(wid * nch, nch)], didx)
        pltpu.sync_copy(msg_hbm.at[pl.ds(wid * epw, epw)], buf)
        _fill_rows(zbuf, RPS, 0.0)
        pltpu.sync_copy(zbuf, agg_sh.at[pl.ds(s * RPS, RPS)])
        plsc.subcore_barrier()

        def sadd(j, wait=False):
            d = pltpu.make_async_copy(buf.at[pl.ds(j * CH, CH)],
                                      agg_sh.at[didx.at[j]], asem)
            d.wait() if wait else d.start(add=True)
        _fire_drain(nch, sadd)

        plsc.subcore_barrier()
        pltpu.sync_copy(agg_sh.at[pl.ds(s * RPS, RPS)],
                        agg_out.at[c].at[pl.ds(s * RPS, RPS)])
    return scatter


EA = 3 * EB
EBP = E - EA
_GATHER_F = _make_gather(E)
_GATHER_A = _make_gather(EA)
_GATHER_B = _make_gather(EBP)
_SCATTER_A = _make_scatter(EA)
_SCATTER_B = _make_scatter(EBP)


def _dense_body(e_ref, xs_ref, nodes_ref, gamma_ref, beta_ref, fold_ref,
                w1_ref, b1_ref, w2_ref, b2_ref, w3_ref, b3_ref,
                r_ref, s_ref, msg_ref, ac_s):
    @pl.when(pl.program_id(0) == 0)
    def _():
        nod = nodes_ref[...]
        ssum = jnp.dot(jnp.sum(nod, axis=0, keepdims=True), fold_ref[...],
                       preferred_element_type=jnp.float32)
        ssq = jnp.dot(jnp.sum(nod * nod, axis=0, keepdims=True), fold_ref[...],
                      preferred_element_type=jnp.float32)
        mu = ssum / float(N)
        var = ssq / float(N) - mu * mu
        a = gamma_ref[...] / jnp.sqrt(var + 1e-5)
        ac_s[...] = jnp.concatenate([a, beta_ref[...] - mu * a], axis=0)

    a = ac_s[0:1, :]
    c = ac_s[1:2, :]
    for g in range(8):
        ecol = e_ref[:, g:g + 1]
        h1 = jnp.maximum(ecol * w1_ref[...] + b1_ref[...], 0.0)
        h2 = jnp.maximum(
            jnp.dot(h1.astype(jnp.bfloat16), w2_ref[...],
                    preferred_element_type=jnp.float32) + b2_ref[...], 0.0)
        w = (jnp.dot(h2.astype(jnp.bfloat16), w3_ref[...],
                     preferred_element_type=jnp.float32) + b3_ref[...])
        xg = xs_ref[:, g * D:(g + 1) * D]
        xn = xg * a + c
        xr = jnp.dot(xn, r_ref[...], preferred_element_type=jnp.float32)
        msg_ref[:, g * D:(g + 1) * D] = jnp.dot(
            xr * w, s_ref[...], preferred_element_type=jnp.float32)


def _dense(nblk, boff, xoff, e_t, xs_p, nodes_p, gamma2, beta2, fold,
           w1, b1r, w2b, b2r, w3b, b3r, r, s):
    full = lambda shape: pl.BlockSpec(shape, lambda i: (0, 0))
    return pl.pallas_call(
        _dense_body,
        grid=(nblk,),
        in_specs=[
            pl.BlockSpec((GR, 8), lambda i: (i + boff, 0)),
            pl.BlockSpec((GR, 128), lambda i: (i + xoff, 0)),
            full((NPK, 128)), full((1, D)), full((1, D)), full((128, D)),
            full((1, H)), full((1, H)),
            full((H, H)), full((1, H)),
            full((H, DD)), full((1, DD)),
            full((D, DD)), full((DD, D)),
        ],
        out_specs=pl.BlockSpec((GR, 128), lambda i: (i, 0)),
        out_shape=jax.ShapeDtypeStruct((nblk * GR, 128), jnp.float32),
        scratch_shapes=[pltpu.VMEM((2, D), jnp.float32)],
    )(e_t, xs_p, nodes_p, gamma2, beta2, fold,
      w1, b1r, w2b, b2r, w3b, b3r, r, s)


def _update_body(agga_ref, aggb_ref, cntp_ref, bias_ref, hin_ref, hout_ref):
    agg = agga_ref[0] + agga_ref[1] + aggb_ref[0] + aggb_ref[1]
    cnt = cntp_ref[0] + cntp_ref[1]
    denom = jnp.maximum(cnt, 1.0)
    hnew = agg / denom + bias_ref[...] + hin_ref[...]
    rowid = lax.broadcasted_iota(jnp.int32, (NPK, 128), 0)
    hout_ref[...] = jnp.where(rowid < NK, hnew, 0.0)


def _update(agga, aggb, cntp, biasp, hinp):
    return pl.pallas_call(
        _update_body,
        out_shape=jax.ShapeDtypeStruct((NPK, 128), jnp.float32),
    )(agga, aggb, cntp, biasp, hinp)


_EYE = np.eye(D, dtype=np.float32)
_R = np.kron(_EYE, np.ones((1, D), np.float32))
_S = np.kron(np.ones((D, 1), np.float32), _EYE)
_FOLD = np.kron(np.ones((8, 1), np.float32), _EYE)
_PERM = np.arange(DD).reshape(D, D).T.reshape(DD)
_S2 = np.kron(_EYE, np.ones((D, 1), np.float32))


def kernel(h, e, edge_index, W1, b1, W2, b2, W3, b3, bias, gamma, beta):
    src2 = edge_index[1].reshape(NW * NCH, CH)
    dst2 = edge_index[0].reshape(NW * NCH, CH)
    e_t = e.reshape(EK, 8)
    hp = jnp.concatenate(
        [h.reshape(NK, 128),
         jnp.zeros((NPK - NK, 128), jnp.float32)], axis=0)
    b1r = b1.reshape(1, H)
    b2r = b2.reshape(1, H)
    b3r = b3.reshape(1, DD)
    biasp = jnp.tile(bias, 8).reshape(1, 128)
    gamma2 = gamma.reshape(1, D)
    beta2 = beta.reshape(1, D)
    w2b = W2.astype(jnp.bfloat16)
    w3b = W3.astype(jnp.bfloat16)
    r = jnp.asarray(_R)
    s = jnp.asarray(_S)
    fold = jnp.asarray(_FOLD)

    rA = EA // CH
    consts = (gamma2, beta2, fold, W1, b1r, w2b, b2r, w3b, b3r, r, s)
    hx = hp.reshape(NP, D)
    cntk = lambda x: x.reshape(NC, NPK, 128)

    xs1 = _GATHER_F(hx, src2)
    cntp = _sc_counts(dst2)
    xs1k = xs1.reshape(EK, 128)
    msg1a = _dense(3, 0, 0, e_t, xs1k, hp, *consts)
    msg1b = _dense(2, 3, 3, e_t, xs1k, hp, *consts)
    agg1a = _SCATTER_A(dst2[:rA], msg1a.reshape(EA, D))
    agg1b = _SCATTER_B(dst2[rA:], msg1b.reshape(EBP, D))
    h2p = _update(cntk(agg1a), cntk(agg1b), cntk(cntp), biasp, hp)
    h2x = h2p.reshape(NP, D)
    xs2a = _GATHER_A(h2x, src2[:rA])
    xs2b = _GATHER_B(h2x, src2[rA:])
    msg2a = _dense(3, 0, 0, e_t, xs2a.reshape(EA // 8, 128), h2p, *consts)
    msg2b = _dense(2, 3, 0, e_t, xs2b.reshape(EBP // 8, 128), h2p, *consts)
    agg2a = _SCATTER_A(dst2[:rA], msg2a.reshape(EA, D))
    agg2b = _SCATTER_B(dst2[rA:], msg2b.reshape(EBP, D))
    h3p = _update(cntk(agg2a), cntk(agg2b), cntk(cntp), biasp, h2p)
    return h3p[:NK].reshape(N, D)


# speedup vs baseline: 1.0959x; 1.0959x over previous
import functools

import numpy as np

import jax
import jax.numpy as jnp
from jax import lax
from jax.experimental import pallas as pl
from jax.experimental.pallas import tpu as pltpu
from jax.experimental.pallas import tpu_sc as plsc

N = 10000
E = 160000
D = 16
H = 128
DD = D * D

NC = 2
NS = 16
NW = NC * NS
EPW = E // NW
CH = 125
NCH = EPW // CH
NP = 10240
RPS = NP // NS
NPK = NP // 8
NK = N // 8

EB = 32000
GR = EB // 8
NB = E // EB
EK = E // 8

_MESH = plsc.VectorSubcoreMesh(core_axis_name="c", subcore_axis_name="s")
_SC_PARAMS = pltpu.CompilerParams(use_tc_tiling_on_sc=False)


def _fill_rows(ref, n, value):
    def body(i, carry):
        ref[i, :] = jnp.full((D,), value, jnp.float32)
        return carry
    lax.fori_loop(0, n, body, 0)


def _fire_drain(n, fire):
    def fire_body(j, carry):
        fire(j)
        return carry
    lax.fori_loop(0, n, fire_body, 0)

    def drain_body(j, carry):
        fire(0, wait=True)
        return carry
    lax.fori_loop(0, n, drain_body, 0)


@functools.partial(
    pl.kernel,
    out_type=jax.ShapeDtypeStruct((NC, NP, D), jnp.float32),
    scratch_types=[
        pltpu.VMEM((NCH, CH), jnp.int32),
        pltpu.VMEM((CH, D), jnp.float32),
        pltpu.VMEM((RPS, D), jnp.float32),
        pltpu.SemaphoreType.DMA,
        pltpu.VMEM_SHARED((NP, D), jnp.float32),
    ],
    mesh=_MESH,
    compiler_params=_SC_PARAMS,
)
def _sc_counts(dst_hbm, cnt_out, didx, obuf, zbuf, csem, cnt_sh):
    c = lax.axis_index("c")
    s = lax.axis_index("s")
    wid = s * NC + c
    pltpu.sync_copy(dst_hbm.at[pl.ds(wid * NCH, NCH)], didx)
    _fill_rows(obuf, CH, 1.0)
    _fill_rows(zbuf, RPS, 0.0)
    pltpu.sync_copy(zbuf, cnt_sh.at[pl.ds(s * RPS, RPS)])
    plsc.subcore_barrier()

    def cadd(j, wait=False):
        d = pltpu.make_async_copy(obuf, cnt_sh.at[didx.at[j]], csem)
        d.wait() if wait else d.start(add=True)
    _fire_drain(NCH, cadd)

    plsc.subcore_barrier()
    pltpu.sync_copy(cnt_sh.at[pl.ds(s * RPS, RPS)],
                    cnt_out.at[c].at[pl.ds(s * RPS, RPS)])


def _make_gather(ep):
    epw = ep // NW
    nch = epw // CH

    @functools.partial(
        pl.kernel,
        out_type=jax.ShapeDtypeStruct((ep, D), jnp.float32),
        scratch_types=[
            pltpu.VMEM((nch, CH), jnp.int32),
            pltpu.VMEM((epw, D), jnp.float32),
            pltpu.SemaphoreType.DMA,
        ],
        mesh=_MESH,
        compiler_params=_SC_PARAMS,
    )
    def gather(x_hbm, src_hbm, xs_out, sidx, rows, gsem):
        c = lax.axis_index("c")
        s = lax.axis_index("s")
        wid = s * NC + c
        pltpu.sync_copy(src_hbm.at[pl.ds(wid * nch, nch)], sidx)

        def gath(j, wait=False):
            d = pltpu.make_async_copy(x_hbm.at[sidx.at[j]],
                                      rows.at[pl.ds(j * CH, CH)], gsem)
            d.wait() if wait else d.start()
        _fire_drain(nch, gath)

        pltpu.sync_copy(rows, xs_out.at[pl.ds(wid * epw, epw)])
    return gather


def _make_scatter(ep):
    epw = ep // NW
    nch = epw // CH

    @functools.partial(
        pl.kernel,
        out_type=jax.ShapeDtypeStruct((NC, NP, D), jnp.float32),
        scratch_types=[
            pltpu.VMEM((nch, CH), jnp.int32),
            pltpu.VMEM((epw, D), jnp.float32),
            pltpu.VMEM((RPS, D), jnp.float32),
            pltpu.SemaphoreType.DMA,
            pltpu.VMEM_SHARED((NP, D), jnp.float32),
        ],
        mesh=_MESH,
        compiler_params=_SC_PARAMS,
    )
    def scatter(dst_hbm, msg_hbm, agg_out, didx, buf, zbuf, asem, agg_sh):
        c = lax.axis_index("c")
        s = lax.axis_index("s")
        wid = s * NC + c
        pltpu.sync_copy(dst_hbm.at[pl.ds(wid * nch, nch)], didx)
        pltpu.sync_copy(msg_hbm.at[pl.ds(wid * epw, epw)], buf)
        _fill_rows(zbuf, RPS, 0.0)
        pltpu.sync_copy(zbuf, agg_sh.at[pl.ds(s * RPS, RPS)])
        plsc.subcore_barrier()

        def sadd(j, wait=False):
            d = pltpu.make_async_copy(buf.at[pl.ds(j * CH, CH)],
                                      agg_sh.at[didx.at[j]], asem)
            d.wait() if wait else d.start(add=True)
        _fire_drain(nch, sadd)

        plsc.subcore_barrier()
        pltpu.sync_copy(agg_sh.at[pl.ds(s * RPS, RPS)],
                        agg_out.at[c].at[pl.ds(s * RPS, RPS)])
    return scatter


_GATHER_F = _make_gather(E)
_SCATTER_F = _make_scatter(E)


def _dense_body(e_ref, xs_ref, nodes_ref, gamma_ref, beta_ref, fold_ref,
                w1_ref, b1_ref, w2_ref, b2_ref, w3_ref, b3_ref,
                r_ref, s_ref, msg_ref, ac_s):
    @pl.when(pl.program_id(0) == 0)
    def _():
        nod = nodes_ref[...]
        ssum = jnp.dot(jnp.sum(nod, axis=0, keepdims=True), fold_ref[...],
                       preferred_element_type=jnp.float32)
        ssq = jnp.dot(jnp.sum(nod * nod, axis=0, keepdims=True), fold_ref[...],
                      preferred_element_type=jnp.float32)
        mu = ssum / float(N)
        var = ssq / float(N) - mu * mu
        a = gamma_ref[...] / jnp.sqrt(var + 1e-5)
        ac_s[...] = jnp.concatenate([a, beta_ref[...] - mu * a], axis=0)

    a = ac_s[0:1, :]
    c = ac_s[1:2, :]
    for g in range(8):
        ecol = e_ref[:, g:g + 1]
        h1 = jnp.maximum(ecol * w1_ref[...] + b1_ref[...], 0.0)
        h2 = jnp.maximum(
            jnp.dot(h1.astype(jnp.bfloat16), w2_ref[...],
                    preferred_element_type=jnp.float32) + b2_ref[...], 0.0)
        w = (jnp.dot(h2.astype(jnp.bfloat16), w3_ref[...],
                     preferred_element_type=jnp.float32) + b3_ref[...])
        xg = xs_ref[:, g * D:(g + 1) * D]
        xn = xg * a + c
        xr = jnp.dot(xn, r_ref[...], preferred_element_type=jnp.float32)
        msg_ref[:, g * D:(g + 1) * D] = jnp.dot(
            xr * w, s_ref[...], preferred_element_type=jnp.float32)


def _dense(nblk, boff, xoff, e_t, xs_p, nodes_p, gamma2, beta2, fold,
           w1, b1r, w2b, b2r, w3b, b3r, r, s):
    full = lambda shape: pl.BlockSpec(shape, lambda i: (0, 0))
    return pl.pallas_call(
        _dense_body,
        grid=(nblk,),
        in_specs=[
            pl.BlockSpec((GR, 8), lambda i: (i + boff, 0)),
            pl.BlockSpec((GR, 128), lambda i: (i + xoff, 0)),
            full((NPK, 128)), full((1, D)), full((1, D)), full((128, D)),
            full((1, H)), full((1, H)),
            full((H, H)), full((1, H)),
            full((H, DD)), full((1, DD)),
            full((D, DD)), full((DD, D)),
        ],
        out_specs=pl.BlockSpec((GR, 128), lambda i: (i, 0)),
        out_shape=jax.ShapeDtypeStruct((nblk * GR, 128), jnp.float32),
        scratch_shapes=[pltpu.VMEM((2, D), jnp.float32)],
    )(e_t, xs_p, nodes_p, gamma2, beta2, fold,
      w1, b1r, w2b, b2r, w3b, b3r, r, s)


def _update_body(aggp_ref, cntp_ref, bias_ref, hin_ref, hout_ref):
    agg = aggp_ref[0] + aggp_ref[1]
    cnt = cntp_ref[0] + cntp_ref[1]
    denom = jnp.maximum(cnt, 1.0)
    hnew = agg / denom + bias_ref[...] + hin_ref[...]
    rowid = lax.broadcasted_iota(jnp.int32, (NPK, 128), 0)
    hout_ref[...] = jnp.where(rowid < NK, hnew, 0.0)


def _update(aggp, cntp, biasp, hinp):
    return pl.pallas_call(
        _update_body,
        out_shape=jax.ShapeDtypeStruct((NPK, 128), jnp.float32),
    )(aggp, cntp, biasp, hinp)


_EYE = np.eye(D, dtype=np.float32)
_R = np.kron(_EYE, np.ones((1, D), np.float32))
_S = np.kron(np.ones((D, 1), np.float32), _EYE)
_FOLD = np.kron(np.ones((8, 1), np.float32), _EYE)
_PERM = np.arange(DD).reshape(D, D).T.reshape(DD)
_S2 = np.kron(_EYE, np.ones((D, 1), np.float32))


def kernel(h, e, edge_index, W1, b1, W2, b2, W3, b3, bias, gamma, beta):
    src2 = edge_index[1].reshape(NW * NCH, CH)
    dst2 = edge_index[0].reshape(NW * NCH, CH)
    e_t = e.reshape(EK, 8)
    hp = jnp.concatenate(
        [h.reshape(NK, 128),
         jnp.zeros((NPK - NK, 128), jnp.float32)], axis=0)
    b1r = b1.reshape(1, H)
    b2r = b2.reshape(1, H)
    b3r = b3.reshape(1, DD)
    biasp = jnp.tile(bias, 8).reshape(1, 128)
    gamma2 = gamma.reshape(1, D)
    beta2 = beta.reshape(1, D)
    w2b = W2.astype(jnp.bfloat16)
    w3b = W3.astype(jnp.bfloat16)
    r = jnp.asarray(_R)
    s = jnp.asarray(_S)
    fold = jnp.asarray(_FOLD)

    consts = (gamma2, beta2, fold, W1, b1r, w2b, b2r, w3b, b3r, r, s)
    hx = hp.reshape(NP, D)
    cntk = lambda x: x.reshape(NC, NPK, 128)

    xs1 = _GATHER_F(hx, src2)
    cntp = _sc_counts(dst2)
    msg1 = _dense(NB, 0, 0, e_t, xs1.reshape(EK, 128), hp, *consts)
    agg1 = _SCATTER_F(dst2, msg1.reshape(E, D))
    h2p = _update(cntk(agg1), cntk(cntp), biasp, hp)
    xs2 = _GATHER_F(h2p.reshape(NP, D), src2)
    msg2 = _dense(NB, 0, 0, e_t, xs2.reshape(EK, 128), h2p, *consts)
    agg2 = _SCATTER_F(dst2, msg2.reshape(E, D))
    h3p = _update(cntk(agg2), cntk(cntp), biasp, h2p)
    return h3p[:NK].reshape(N, D)
